# Initial kernel scaffold; baseline (speedup 1.0000x reference)
import jax
import jax.numpy as jnp
from jax.experimental import pallas as pl


def kernel(dataset, edge_index, W1, b1, W2, b2):
    raise NotImplementedError("write your pallas kernel here")


# speedup vs baseline: 8.6055x; 8.6055x over previous
import functools

import jax
import jax.numpy as jnp
from jax import lax
from jax.experimental import pallas as pl
from jax.experimental.pallas import tpu as pltpu
from jax.experimental.pallas import tpu_sc as plsc

N = 10000
E = 320000
DIN = 128
DH = 16
DO = 10
DF = 128

NC = 2
NS = 16
NW = NC * NS
CH = 128
NCH = 80
EPW = CH * NCH
EPAD = EPW * NW
NPAD = 10112
RPT = NPAD // NS

_mesh = plsc.VectorSubcoreMesh(core_axis_name="c", subcore_axis_name="s")


@functools.partial(
    pl.kernel,
    out_type=jax.ShapeDtypeStruct((NC, NPAD, DF), jnp.float32),
    mesh=_mesh,
    scratch_types=[
        pltpu.VMEM((NCH, CH), jnp.int32),
        pltpu.VMEM((CH, DF), jnp.float32),
        pltpu.VMEM_SHARED((NPAD, DF), jnp.float32),
    ],
)
def _sc_degree(colp_hbm, ones_hbm, zeros_hbm, out_hbm, idx_c, ones_v, acc):
    c = lax.axis_index("c")
    s = lax.axis_index("s")
    wid = s * NC + c
    pltpu.sync_copy(zeros_hbm, acc.at[pl.ds(s * RPT, RPT)])
    pltpu.sync_copy(colp_hbm.at[wid], idx_c)
    pltpu.sync_copy(ones_hbm, ones_v)
    plsc.subcore_barrier()

    def body(j, carry):
        pltpu.sync_copy(ones_v, acc.at[idx_c.at[j]], add=True)
        return carry

    lax.fori_loop(0, NCH, body, 0)
    plsc.subcore_barrier()
    pltpu.sync_copy(acc.at[pl.ds(s * RPT, RPT)],
                    out_hbm.at[c, pl.ds(s * RPT, RPT)])


@functools.partial(
    pl.kernel,
    out_type=jax.ShapeDtypeStruct((NC, NPAD, DF), jnp.float32),
    mesh=_mesh,
    scratch_types=[
        pltpu.VMEM((NCH, CH), jnp.int32),
        pltpu.VMEM((NCH, CH), jnp.int32),
        pltpu.VMEM((CH, DF), jnp.float32),
        pltpu.VMEM_SHARED((NPAD, DF), jnp.float32),
        pltpu.SemaphoreType.DMA,
    ],
)
def _sc_scatter(g_hbm, rowp_hbm, colp_hbm, zeros_hbm, out_hbm,
                idx_r, idx_c, buf, acc, sem):
    c = lax.axis_index("c")
    s = lax.axis_index("s")
    wid = s * NC + c
    pltpu.sync_copy(zeros_hbm, acc.at[pl.ds(s * RPT, RPT)])
    pltpu.sync_copy(rowp_hbm.at[wid], idx_r)
    pltpu.sync_copy(colp_hbm.at[wid], idx_c)
    plsc.subcore_barrier()

    def body(j, carry):
        pltpu.async_copy(g_hbm.at[idx_r.at[j]], buf, sem).wait()
        pltpu.sync_copy(buf, acc.at[idx_c.at[j]], add=True)
        return carry

    lax.fori_loop(0, NCH, body, 0)
    plsc.subcore_barrier()
    pltpu.sync_copy(acc.at[pl.ds(s * RPT, RPT)],
                    out_hbm.at[c, pl.ds(s * RPT, RPT)])


_BLK = 1000


def _tc1_body(degt_ref, x_ref, w_ref, g_ref):
    deg = degt_ref[:, 0:1] + degt_ref[:, 1:2] + 1.0
    dinv = lax.rsqrt(deg)
    h = jnp.dot(x_ref[...], w_ref[...], preferred_element_type=jnp.float32)
    g_ref[...] = h * dinv


def _tc2_body(degt_ref, p0_ref, p1_ref, g1_ref, b1_ref, w2_ref, g2_ref):
    deg = degt_ref[:, 0:1] + degt_ref[:, 1:2] + 1.0
    dinv = lax.rsqrt(deg)
    pre = dinv * (p0_ref[...] + p1_ref[...] + g1_ref[...]) + b1_ref[...]
    o1 = jnp.maximum(pre, 0.0)
    h2 = jnp.dot(o1, w2_ref[...], preferred_element_type=jnp.float32)
    g2_ref[...] = h2 * dinv


def _tc3_body(degt_ref, q0_ref, q1_ref, g2_ref, b2_ref, z_ref):
    deg = degt_ref[:, 0:1] + degt_ref[:, 1:2] + 1.0
    dinv = lax.rsqrt(deg)
    z = dinv * (q0_ref[...] + q1_ref[...] + g2_ref[...]) + b2_ref[...]
    mask = lax.broadcasted_iota(jnp.int32, z.shape, 1) < DO
    zm = jnp.where(mask, z, -jnp.inf)
    m = jnp.max(zm, axis=1, keepdims=True)
    ex = jnp.where(mask, jnp.exp(z - m), 0.0)
    ssum = jnp.sum(ex, axis=1, keepdims=True)
    z_ref[...] = z - m - jnp.log(ssum)


def _row_spec(width):
    return pl.BlockSpec((_BLK, width), lambda i: (i, 0))


def _full_spec(shape):
    return pl.BlockSpec(shape, lambda i: (0, 0))


def _tc1(degt, x, w1):
    return pl.pallas_call(
        _tc1_body,
        grid=(N // _BLK,),
        in_specs=[_row_spec(2), _row_spec(DIN), _full_spec((DIN, DF))],
        out_specs=_row_spec(DF),
        out_shape=jax.ShapeDtypeStruct((N, DF), jnp.float32),
    )(degt, x, w1)


def _tc2(degt, p0, p1, g1, b1, w2):
    return pl.pallas_call(
        _tc2_body,
        grid=(N // _BLK,),
        in_specs=[_row_spec(2), _row_spec(DF), _row_spec(DF), _row_spec(DF),
                  _full_spec((1, DF)), _full_spec((DF, DF))],
        out_specs=_row_spec(DF),
        out_shape=jax.ShapeDtypeStruct((N, DF), jnp.float32),
    )(degt, p0, p1, g1, b1, w2)


def _tc3(degt, q0, q1, g2, b2):
    return pl.pallas_call(
        _tc3_body,
        grid=(N // _BLK,),
        in_specs=[_row_spec(2), _row_spec(DF), _row_spec(DF), _row_spec(DF),
                  _full_spec((1, DF))],
        out_specs=_row_spec(DF),
        out_shape=jax.ShapeDtypeStruct((N, DF), jnp.float32),
    )(degt, q0, q1, g2, b2)


def kernel(dataset, edge_index, W1, b1, W2, b2):
    ei = edge_index.astype(jnp.int32)
    row = ei[0]
    col = ei[1]
    pad = EPAD - E
    rowp = jnp.concatenate([row, jnp.zeros((pad,), jnp.int32)])
    colp = jnp.concatenate([col, jnp.full((pad,), N, jnp.int32)])
    rowp = rowp.reshape(NW, NCH, CH)
    colp = colp.reshape(NW, NCH, CH)
    zrows = jnp.zeros((RPT, DF), jnp.float32)
    ones = jnp.ones((CH, DF), jnp.float32)

    degp = _sc_degree(colp, ones, zrows)
    degt = jnp.stack([degp[0, :N, 0], degp[1, :N, 0]], axis=1)

    w1p = jnp.pad(W1, ((0, 0), (0, DF - DH)))
    g1 = _tc1(degt, dataset, w1p)
    g1p = jnp.pad(g1, ((0, NPAD - N), (0, 0)))
    s1 = _sc_scatter(g1p, rowp, colp, zrows)
    b1p = jnp.pad(b1, (0, DF - DH)).reshape(1, DF)
    w2p = jnp.pad(W2, ((0, DF - DH), (0, DF - DO)))
    g2 = _tc2(degt, s1[0, :N], s1[1, :N], g1, b1p, w2p)
    g2p = jnp.pad(g2, ((0, NPAD - N), (0, 0)))
    s2 = _sc_scatter(g2p, rowp, colp, zrows)
    b2p = jnp.pad(b2, (0, DF - DO)).reshape(1, DF)
    z = _tc3(degt, s2[0, :N], s2[1, :N], g2, b2p)
    return z[:, :DO]


# speedup vs baseline: 8.7916x; 1.0216x over previous
import functools

import jax
import jax.numpy as jnp
from jax import lax
from jax.experimental import pallas as pl
from jax.experimental.pallas import tpu as pltpu
from jax.experimental.pallas import tpu_sc as plsc

N = 10000
E = 320000
DIN = 128
DH = 16
DO = 10
DF = 128

NC = 2
NS = 16
NW = NC * NS
CH = 128
NCH = 80
EPW = CH * NCH
EPAD = EPW * NW
NPAD = 10112
RPT = NPAD // NS

_mesh = plsc.VectorSubcoreMesh(core_axis_name="c", subcore_axis_name="s")


@functools.partial(
    pl.kernel,
    out_type=jax.ShapeDtypeStruct((NC, NPAD, DF), jnp.float32),
    mesh=_mesh,
    scratch_types=[
        pltpu.VMEM((NCH, CH), jnp.int32),
        pltpu.VMEM((CH, DF), jnp.float32),
        pltpu.VMEM_SHARED((NPAD, DF), jnp.float32),
    ],
)
def _sc_degree(colp_hbm, ones_hbm, zeros_hbm, out_hbm, idx_c, ones_v, acc):
    c = lax.axis_index("c")
    s = lax.axis_index("s")
    wid = s * NC + c
    pltpu.sync_copy(zeros_hbm, acc.at[pl.ds(s * RPT, RPT)])
    pltpu.sync_copy(colp_hbm.at[wid], idx_c)
    pltpu.sync_copy(ones_hbm, ones_v)
    plsc.subcore_barrier()

    def body(j, carry):
        pltpu.sync_copy(ones_v, acc.at[idx_c.at[j]], add=True)
        return carry

    lax.fori_loop(0, NCH, body, 0)
    plsc.subcore_barrier()
    pltpu.sync_copy(acc.at[pl.ds(s * RPT, RPT)],
                    out_hbm.at[c, pl.ds(s * RPT, RPT)])


@functools.partial(
    pl.kernel,
    out_type=jax.ShapeDtypeStruct((NC, NPAD, DF), jnp.float32),
    mesh=_mesh,
    scratch_types=[
        pltpu.VMEM((NCH // 2, CH), jnp.int32),
        pltpu.VMEM((NCH // 2, CH), jnp.int32),
        pltpu.VMEM((CH, DF), jnp.float32),
        pltpu.VMEM((CH, DF), jnp.float32),
        pltpu.VMEM_SHARED((NPAD, DF), jnp.float32),
        pltpu.SemaphoreType.DMA,
        pltpu.SemaphoreType.DMA,
    ],
)
def _sc_scatter(g_hbm, rowp_hbm, colp_hbm, zeros_hbm, out_hbm,
                idx_r, idx_c, b0, b1, acc, s0, s1):
    c = lax.axis_index("c")
    s = lax.axis_index("s")
    wid = s * NC + c
    hc = NCH // 2
    pltpu.sync_copy(zeros_hbm, acc.at[pl.ds(s * RPT, RPT)])
    plsc.subcore_barrier()

    def half(h, carry):
        pltpu.sync_copy(rowp_hbm.at[wid, pl.ds(h * hc, hc)], idx_r)
        pltpu.sync_copy(colp_hbm.at[wid, pl.ds(h * hc, hc)], idx_c)

        def body(k, carry2):
            j = 2 * k
            g0 = pltpu.async_copy(g_hbm.at[idx_r.at[j]], b0, s0)
            g1 = pltpu.async_copy(g_hbm.at[idx_r.at[j + 1]], b1, s1)
            g0.wait()
            pltpu.sync_copy(b0, acc.at[idx_c.at[j]], add=True)
            g1.wait()
            pltpu.sync_copy(b1, acc.at[idx_c.at[j + 1]], add=True)
            return carry2

        lax.fori_loop(0, hc // 2, body, 0)
        return carry

    lax.fori_loop(0, 2, half, 0)
    plsc.subcore_barrier()
    pltpu.sync_copy(acc.at[pl.ds(s * RPT, RPT)],
                    out_hbm.at[c, pl.ds(s * RPT, RPT)])


_BLK = 1000


def _tc1_body(degt_ref, x_ref, w_ref, g_ref):
    deg = degt_ref[:, 0:1] + degt_ref[:, 1:2] + 1.0
    dinv = lax.rsqrt(deg)
    h = jnp.dot(x_ref[...], w_ref[...], preferred_element_type=jnp.float32)
    g_ref[...] = h * dinv


def _tc2_body(degt_ref, p0_ref, p1_ref, g1_ref, b1_ref, w2_ref, g2_ref):
    deg = degt_ref[:, 0:1] + degt_ref[:, 1:2] + 1.0
    dinv = lax.rsqrt(deg)
    pre = dinv * (p0_ref[...] + p1_ref[...] + g1_ref[...]) + b1_ref[...]
    o1 = jnp.maximum(pre, 0.0)
    h2 = jnp.dot(o1, w2_ref[...], preferred_element_type=jnp.float32)
    g2_ref[...] = h2 * dinv


def _tc3_body(degt_ref, q0_ref, q1_ref, g2_ref, b2_ref, z_ref):
    deg = degt_ref[:, 0:1] + degt_ref[:, 1:2] + 1.0
    dinv = lax.rsqrt(deg)
    z = dinv * (q0_ref[...] + q1_ref[...] + g2_ref[...]) + b2_ref[...]
    mask = lax.broadcasted_iota(jnp.int32, z.shape, 1) < DO
    zm = jnp.where(mask, z, -jnp.inf)
    m = jnp.max(zm, axis=1, keepdims=True)
    ex = jnp.where(mask, jnp.exp(z - m), 0.0)
    ssum = jnp.sum(ex, axis=1, keepdims=True)
    z_ref[...] = z - m - jnp.log(ssum)


def _row_spec(width):
    return pl.BlockSpec((_BLK, width), lambda i: (i, 0))


def _full_spec(shape):
    return pl.BlockSpec(shape, lambda i: (0, 0))


def _tc1(degt, x, w1):
    return pl.pallas_call(
        _tc1_body,
        grid=(N // _BLK,),
        in_specs=[_row_spec(2), _row_spec(DIN), _full_spec((DIN, DF))],
        out_specs=_row_spec(DF),
        out_shape=jax.ShapeDtypeStruct((N, DF), jnp.float32),
    )(degt, x, w1)


def _tc2(degt, p0, p1, g1, b1, w2):
    return pl.pallas_call(
        _tc2_body,
        grid=(N // _BLK,),
        in_specs=[_row_spec(2), _row_spec(DF), _row_spec(DF), _row_spec(DF),
                  _full_spec((1, DF)), _full_spec((DF, DF))],
        out_specs=_row_spec(DF),
        out_shape=jax.ShapeDtypeStruct((N, DF), jnp.float32),
    )(degt, p0, p1, g1, b1, w2)


def _tc3(degt, q0, q1, g2, b2):
    return pl.pallas_call(
        _tc3_body,
        grid=(N // _BLK,),
        in_specs=[_row_spec(2), _row_spec(DF), _row_spec(DF), _row_spec(DF),
                  _full_spec((1, DF))],
        out_specs=_row_spec(DF),
        out_shape=jax.ShapeDtypeStruct((N, DF), jnp.float32),
    )(degt, q0, q1, g2, b2)


def kernel(dataset, edge_index, W1, b1, W2, b2):
    ei = edge_index.astype(jnp.int32)
    row = ei[0]
    col = ei[1]
    pad = EPAD - E
    rowp = jnp.concatenate([row, jnp.zeros((pad,), jnp.int32)])
    colp = jnp.concatenate([col, jnp.full((pad,), N, jnp.int32)])
    rowp = rowp.reshape(NW, NCH, CH)
    colp = colp.reshape(NW, NCH, CH)
    zrows = jnp.zeros((RPT, DF), jnp.float32)
    ones = jnp.ones((CH, DF), jnp.float32)

    degp = _sc_degree(colp, ones, zrows)
    degt = jnp.stack([degp[0, :N, 0], degp[1, :N, 0]], axis=1)

    w1p = jnp.pad(W1, ((0, 0), (0, DF - DH)))
    g1 = _tc1(degt, dataset, w1p)
    g1p = jnp.pad(g1, ((0, NPAD - N), (0, 0)))
    s1 = _sc_scatter(g1p, rowp, colp, zrows)
    b1p = jnp.pad(b1, (0, DF - DH)).reshape(1, DF)
    w2p = jnp.pad(W2, ((0, DF - DH), (0, DF - DO)))
    g2 = _tc2(degt, s1[0, :N], s1[1, :N], g1, b1p, w2p)
    g2p = jnp.pad(g2, ((0, NPAD - N), (0, 0)))
    s2 = _sc_scatter(g2p, rowp, colp, zrows)
    b2p = jnp.pad(b2, (0, DF - DO)).reshape(1, DF)
    z = _tc3(degt, s2[0, :N], s2[1, :N], g2, b2p)
    return z[:, :DO]


# speedup vs baseline: 9.5322x; 1.0842x over previous
import functools

import jax
import jax.numpy as jnp
from jax import lax
from jax.experimental import pallas as pl
from jax.experimental.pallas import tpu as pltpu
from jax.experimental.pallas import tpu_sc as plsc

N = 10000
E = 320000
DIN = 128
DH = 16
DO = 10
DF = 128

NC = 2
NS = 16
NW = NC * NS
CH = 128
NCH = 80
EPW = CH * NCH
EPAD = EPW * NW
NPAD = 10112
RPT = NPAD // NS

_mesh = plsc.VectorSubcoreMesh(core_axis_name="c", subcore_axis_name="s")


@functools.partial(
    pl.kernel,
    out_type=jax.ShapeDtypeStruct((NC, NPAD, DF), jnp.float32),
    mesh=_mesh,
    scratch_types=[
        pltpu.VMEM((NCH, CH), jnp.int32),
        pltpu.VMEM((CH, DF), jnp.float32),
        pltpu.VMEM_SHARED((NPAD, DF), jnp.float32),
    ],
)
def _sc_degree(colp_hbm, ones_hbm, zeros_hbm, out_hbm, idx_c, ones_v, acc):
    c = lax.axis_index("c")
    s = lax.axis_index("s")
    wid = s * NC + c
    pltpu.sync_copy(zeros_hbm, acc.at[pl.ds(s * RPT, RPT)])
    pltpu.sync_copy(colp_hbm.at[wid], idx_c)
    pltpu.sync_copy(ones_hbm, ones_v)
    plsc.subcore_barrier()

    def body(j, carry):
        pltpu.sync_copy(ones_v, acc.at[idx_c.at[j]], add=True)
        return carry

    lax.fori_loop(0, NCH, body, 0)
    plsc.subcore_barrier()
    pltpu.sync_copy(acc.at[pl.ds(s * RPT, RPT)],
                    out_hbm.at[c, pl.ds(s * RPT, RPT)])


QC = 16
NQ = NCH // QC


@functools.partial(
    pl.kernel,
    out_type=jax.ShapeDtypeStruct((NC, NPAD, DF), jnp.float32),
    mesh=_mesh,
    scratch_types=[
        pltpu.VMEM((QC, CH), jnp.int32),
        pltpu.VMEM((QC, CH), jnp.int32),
        pltpu.VMEM((CH, DF), jnp.float32),
        pltpu.VMEM((CH, DF), jnp.float32),
        pltpu.VMEM_SHARED((NPAD, DF), jnp.float32),
        pltpu.SemaphoreType.DMA,
        pltpu.SemaphoreType.DMA,
    ],
)
def _sc_scatter(g_hbm, rowp_hbm, colp_hbm, zeros_hbm, out_hbm,
                idx_r, idx_c, b0, b1, acc, s0, s1):
    c = lax.axis_index("c")
    s = lax.axis_index("s")
    wid = s * NC + c
    pltpu.sync_copy(zeros_hbm, acc.at[pl.ds(s * RPT, RPT)])
    plsc.subcore_barrier()

    def stage(q, carry):
        pltpu.sync_copy(rowp_hbm.at[wid, pl.ds(q * QC, QC)], idx_r)
        pltpu.sync_copy(colp_hbm.at[wid, pl.ds(q * QC, QC)], idx_c)
        pltpu.async_copy(g_hbm.at[idx_r.at[0]], b0, s0)
        pltpu.async_copy(g_hbm.at[idx_r.at[1]], b1, s1)

        def body(k, carry2):
            j = 2 * k
            pltpu.make_async_copy(g_hbm.at[idx_r.at[j]], b0, s0).wait()
            pltpu.sync_copy(b0, acc.at[idx_c.at[j]], add=True)
            pltpu.async_copy(g_hbm.at[idx_r.at[j + 2]], b0, s0)
            pltpu.make_async_copy(g_hbm.at[idx_r.at[j + 1]], b1, s1).wait()
            pltpu.sync_copy(b1, acc.at[idx_c.at[j + 1]], add=True)
            pltpu.async_copy(g_hbm.at[idx_r.at[j + 3]], b1, s1)
            return carry2

        lax.fori_loop(0, QC // 2 - 1, body, 0)
        j = QC - 2
        pltpu.make_async_copy(g_hbm.at[idx_r.at[j]], b0, s0).wait()
        pltpu.sync_copy(b0, acc.at[idx_c.at[j]], add=True)
        pltpu.make_async_copy(g_hbm.at[idx_r.at[j + 1]], b1, s1).wait()
        pltpu.sync_copy(b1, acc.at[idx_c.at[j + 1]], add=True)
        return carry

    lax.fori_loop(0, NQ, stage, 0)
    plsc.subcore_barrier()
    pltpu.sync_copy(acc.at[pl.ds(s * RPT, RPT)],
                    out_hbm.at[c, pl.ds(s * RPT, RPT)])


_BLK = 1000


def _tc1_body(degt_ref, x_ref, w_ref, g_ref):
    deg = degt_ref[:, 0:1] + degt_ref[:, 1:2] + 1.0
    dinv = lax.rsqrt(deg)
    h = jnp.dot(x_ref[...], w_ref[...], preferred_element_type=jnp.float32)
    g_ref[...] = h * dinv


def _tc2_body(degt_ref, p0_ref, p1_ref, g1_ref, b1_ref, w2_ref, g2_ref):
    deg = degt_ref[:, 0:1] + degt_ref[:, 1:2] + 1.0
    dinv = lax.rsqrt(deg)
    pre = dinv * (p0_ref[...] + p1_ref[...] + g1_ref[...]) + b1_ref[...]
    o1 = jnp.maximum(pre, 0.0)
    h2 = jnp.dot(o1, w2_ref[...], preferred_element_type=jnp.float32)
    g2_ref[...] = h2 * dinv


def _tc3_body(degt_ref, q0_ref, q1_ref, g2_ref, b2_ref, z_ref):
    deg = degt_ref[:, 0:1] + degt_ref[:, 1:2] + 1.0
    dinv = lax.rsqrt(deg)
    z = dinv * (q0_ref[...] + q1_ref[...] + g2_ref[...]) + b2_ref[...]
    mask = lax.broadcasted_iota(jnp.int32, z.shape, 1) < DO
    zm = jnp.where(mask, z, -jnp.inf)
    m = jnp.max(zm, axis=1, keepdims=True)
    ex = jnp.where(mask, jnp.exp(z - m), 0.0)
    ssum = jnp.sum(ex, axis=1, keepdims=True)
    z_ref[...] = z - m - jnp.log(ssum)


def _row_spec(width):
    return pl.BlockSpec((_BLK, width), lambda i: (i, 0))


def _full_spec(shape):
    return pl.BlockSpec(shape, lambda i: (0, 0))


def _tc1(degt, x, w1):
    return pl.pallas_call(
        _tc1_body,
        grid=(N // _BLK,),
        in_specs=[_row_spec(2), _row_spec(DIN), _full_spec((DIN, DF))],
        out_specs=_row_spec(DF),
        out_shape=jax.ShapeDtypeStruct((N, DF), jnp.float32),
    )(degt, x, w1)


def _tc2(degt, p0, p1, g1, b1, w2):
    return pl.pallas_call(
        _tc2_body,
        grid=(N // _BLK,),
        in_specs=[_row_spec(2), _row_spec(DF), _row_spec(DF), _row_spec(DF),
                  _full_spec((1, DF)), _full_spec((DF, DF))],
        out_specs=_row_spec(DF),
        out_shape=jax.ShapeDtypeStruct((N, DF), jnp.float32),
    )(degt, p0, p1, g1, b1, w2)


def _tc3(degt, q0, q1, g2, b2):
    return pl.pallas_call(
        _tc3_body,
        grid=(N // _BLK,),
        in_specs=[_row_spec(2), _row_spec(DF), _row_spec(DF), _row_spec(DF),
                  _full_spec((1, DF))],
        out_specs=_row_spec(DF),
        out_shape=jax.ShapeDtypeStruct((N, DF), jnp.float32),
    )(degt, q0, q1, g2, b2)


def kernel(dataset, edge_index, W1, b1, W2, b2):
    ei = edge_index.astype(jnp.int32)
    row = ei[0]
    col = ei[1]
    pad = EPAD - E
    rowp = jnp.concatenate([row, jnp.zeros((pad,), jnp.int32)])
    colp = jnp.concatenate([col, jnp.full((pad,), N, jnp.int32)])
    rowp = rowp.reshape(NW, NCH, CH)
    colp = colp.reshape(NW, NCH, CH)
    zrows = jnp.zeros((RPT, DF), jnp.float32)
    ones = jnp.ones((CH, DF), jnp.float32)

    degp = _sc_degree(colp, ones, zrows)
    degt = jnp.stack([degp[0, :N, 0], degp[1, :N, 0]], axis=1)

    w1p = jnp.pad(W1, ((0, 0), (0, DF - DH)))
    g1 = _tc1(degt, dataset, w1p)
    g1p = jnp.pad(g1, ((0, NPAD - N), (0, 0)))
    s1 = _sc_scatter(g1p, rowp, colp, zrows)
    b1p = jnp.pad(b1, (0, DF - DH)).reshape(1, DF)
    w2p = jnp.pad(W2, ((0, DF - DH), (0, DF - DO)))
    g2 = _tc2(degt, s1[0, :N], s1[1, :N], g1, b1p, w2p)
    g2p = jnp.pad(g2, ((0, NPAD - N), (0, 0)))
    s2 = _sc_scatter(g2p, rowp, colp, zrows)
    b2p = jnp.pad(b2, (0, DF - DO)).reshape(1, DF)
    z = _tc3(degt, s2[0, :N], s2[1, :N], g2, b2p)
    return z[:, :DO]
